# baseline (speedup 1.0000x reference)
import functools

import jax
import jax.numpy as jnp
from jax import lax
from jax.experimental import pallas as pl
from jax.experimental.pallas import tpu as pltpu
from jax.experimental.pallas import tpu_sc as plsc

_BETA = 0.25
_D = 128
_K = 512
_BLK = 1792

_NC = 2
_NS = 16
_NW = _NC * _NS


def _conv(x, w, stride, pad):
    return lax.conv_general_dilated(
        x, w, (stride, stride), ((pad, pad), (pad, pad)),
        dimension_numbers=('NCHW', 'OIHW', 'NCHW'))


def _conv_nhwc(x, w, stride, pad):
    return lax.conv_general_dilated(
        x, jnp.transpose(w, (2, 3, 1, 0)), (stride, stride),
        ((pad, pad), (pad, pad)),
        dimension_numbers=('NHWC', 'HWIO', 'NHWC'))


def _resblock_nhwc(x, w1, w2):
    h = _conv_nhwc(jax.nn.relu(x), w1, 1, 1)
    h = _conv_nhwc(jax.nn.relu(h), w2, 1, 0)
    return x + h


def _convT(x, w, stride, pad_eff):
    return lax.conv_general_dilated(
        x, w, (1, 1), ((pad_eff, pad_eff), (pad_eff, pad_eff)),
        lhs_dilation=(stride, stride),
        dimension_numbers=('NCHW', 'OIHW', 'NCHW'))


def _resblock(x, w1, w2):
    h = _conv(jax.nn.relu(x), w1, 1, 1)
    h = _conv(jax.nn.relu(h), w2, 1, 0)
    return x + h


def _vq_body(flat_ref, cb_ref, fsq_ref, cbsq_ref, idx_ref, loss_ref):
    i = pl.program_id(0)
    fb = flat_ref[...]
    cb = cb_ref[...]
    scores = lax.dot_general(
        fb, cb, (((1,), (1,)), ((), ())),
        preferred_element_type=jnp.float32)
    dists = (fsq_ref[...] - 2.0 * scores) + cbsq_ref[...]
    minv = jnp.min(dists, axis=1, keepdims=True)
    lane = lax.broadcasted_iota(jnp.int32, dists.shape, 1)
    idx_ref[0, 0, :] = jnp.min(
        jnp.where(dists == minv, lane, _K), axis=1)
    part = jnp.sum(minv)

    @pl.when(i == 0)
    def _():
        loss_ref[0, 0] = 0.0

    loss_ref[0, 0] += part


def _vq_argmin(flat, cb, fsq, cbsq):
    n = flat.shape[0]
    nblk = n // _BLK
    idx3, dsum = pl.pallas_call(
        _vq_body,
        grid=(nblk,),
        in_specs=[
            pl.BlockSpec((_BLK, _D), lambda i: (i, 0)),
            pl.BlockSpec((_K, _D), lambda i: (0, 0)),
            pl.BlockSpec((_BLK, 1), lambda i: (i, 0)),
            pl.BlockSpec((1, _K), lambda i: (0, 0)),
        ],
        out_specs=[
            pl.BlockSpec((1, 1, _BLK), lambda i: (i, 0, 0)),
            pl.BlockSpec((1, 1), lambda i: (0, 0), memory_space=pltpu.SMEM),
        ],
        out_shape=[
            jax.ShapeDtypeStruct((nblk, 1, _BLK), jnp.int32),
            jax.ShapeDtypeStruct((1, 1), jnp.float32),
        ],
    )(flat, cb, fsq, cbsq)
    return idx3.reshape(-1), dsum[0, 0]


def _sc_gather(cb, idx):
    n = idx.shape[0]
    bpw = n // _NW
    nch = 7
    ch = bpw // nch
    mesh = plsc.VectorSubcoreMesh(core_axis_name="c", subcore_axis_name="s")

    @functools.partial(
        pl.kernel,
        mesh=mesh,
        out_type=jax.ShapeDtypeStruct((n, _D), jnp.float32),
        scratch_types=[
            pltpu.VMEM((bpw,), jnp.int32),
            pltpu.VMEM((2, ch, _D), jnp.float32),
            pltpu.SemaphoreType.DMA,
            pltpu.SemaphoreType.DMA,
            pltpu.SemaphoreType.DMA,
        ],
    )
    def gather_k(table_hbm, idx_hbm, out_hbm, idx_v, rows_v, gsem, ssem, isem):
        wid = lax.axis_index("s") * _NC + lax.axis_index("c")
        base = wid * bpw
        pltpu.async_copy(idx_hbm.at[pl.ds(base, bpw)], idx_v, isem).wait()
        pltpu.async_copy(table_hbm.at[idx_v.at[pl.ds(0, ch)]], rows_v.at[0], gsem)
        for j in range(nch):
            pltpu.make_async_copy(table_hbm.at[idx_v.at[pl.ds(j * ch, ch)]],
                                  rows_v.at[j % 2], gsem).wait()
            if j > 0:
                pltpu.make_async_copy(rows_v.at[(j - 1) % 2],
                                      out_hbm.at[pl.ds(base + (j - 1) * ch, ch)],
                                      ssem).wait()
            if j + 1 < nch:
                pltpu.async_copy(
                    table_hbm.at[idx_v.at[pl.ds((j + 1) * ch, ch)]],
                    rows_v.at[(j + 1) % 2], gsem)
            pltpu.async_copy(rows_v.at[j % 2],
                             out_hbm.at[pl.ds(base + j * ch, ch)], ssem)
        pltpu.make_async_copy(rows_v.at[(nch - 1) % 2],
                              out_hbm.at[pl.ds(base + (nch - 1) * ch, ch)],
                              ssem).wait()

    return gather_k(cb, idx)


def kernel(x, enc_w1, enc_w2, enc_r1_w1, enc_r1_w2, enc_r2_w1, enc_r2_w2,
           codebook, dec_r1_w1, dec_r1_w2, dec_r2_w1, dec_r2_w2, dec_w1,
           dec_w2):
    ob = lax.optimization_barrier
    xh = jnp.transpose(x, (0, 2, 3, 1))
    h = ob(jax.nn.relu(_conv_nhwc(xh, enc_w1, 2, 1)))
    h = ob(_conv_nhwc(h, enc_w2, 2, 1))
    h = ob(_resblock_nhwc(h, enc_r1_w1, enc_r1_w2))
    z = ob(_resblock_nhwc(h, enc_r2_w1, enc_r2_w2))

    b, hh, ww, c = z.shape
    flat = z.reshape(-1, c)
    fsq = jnp.sum(flat ** 2, axis=1, keepdims=True)
    cbsq = jnp.sum(lax.stop_gradient(codebook) ** 2, axis=1)[None, :]
    idx, dsum = _vq_argmin(flat, codebook, fsq, cbsq)
    z_q = _sc_gather(codebook, idx)
    loss = (1.0 + _BETA) * dsum / (flat.shape[0] * c)

    zq = jnp.transpose(z_q.reshape(b, hh, ww, c), (0, 3, 1, 2))
    d = ob(_resblock(zq, dec_r1_w1, dec_r1_w2))
    d = ob(_resblock(d, dec_r2_w1, dec_r2_w2))
    d = jax.nn.relu(d)
    d = ob(jax.nn.relu(_convT(d, dec_w1, 2, 2)))
    x_tilde = jnp.tanh(_convT(d, dec_w2, 2, 2))
    return x_tilde, loss


# speedup vs baseline: 1.0033x; 1.0033x over previous
import functools

import jax
import jax.numpy as jnp
from jax import lax
from jax.experimental import pallas as pl
from jax.experimental.pallas import tpu as pltpu
from jax.experimental.pallas import tpu_sc as plsc

_BETA = 0.25
_D = 128
_K = 512
_BLK = 1792

_NC = 2
_NS = 16
_NW = _NC * _NS


def _conv(x, w, stride, pad):
    return lax.conv_general_dilated(
        x, w, (stride, stride), ((pad, pad), (pad, pad)),
        dimension_numbers=('NCHW', 'OIHW', 'NCHW'))


def _conv_nhwc(x, w, stride, pad):
    return lax.conv_general_dilated(
        x, jnp.transpose(w, (2, 3, 1, 0)), (stride, stride),
        ((pad, pad), (pad, pad)),
        dimension_numbers=('NHWC', 'HWIO', 'NHWC'))


def _resblock_nhwc(x, w1, w2):
    h = _conv_nhwc(jax.nn.relu(x), w1, 1, 1)
    h = _conv_nhwc(jax.nn.relu(h), w2, 1, 0)
    return x + h


def _convT(x, w, stride, pad_eff):
    return lax.conv_general_dilated(
        x, w, (1, 1), ((pad_eff, pad_eff), (pad_eff, pad_eff)),
        lhs_dilation=(stride, stride),
        dimension_numbers=('NCHW', 'OIHW', 'NCHW'))


def _resblock(x, w1, w2):
    h = _conv(jax.nn.relu(x), w1, 1, 1)
    h = _conv(jax.nn.relu(h), w2, 1, 0)
    return x + h


def _vq_body(flat_ref, cb_ref, fsq_ref, cbsq_ref, idx_ref, zq_ref):
    fb = flat_ref[...]
    cb = cb_ref[...]
    scores = lax.dot_general(
        fb, cb, (((1,), (1,)), ((), ())),
        preferred_element_type=jnp.float32)
    dists = (fsq_ref[...] - 2.0 * scores) + cbsq_ref[...]
    minv = jnp.min(dists, axis=1, keepdims=True)
    lane = lax.broadcasted_iota(jnp.int32, dists.shape, 1)
    idx = jnp.min(jnp.where(dists == minv, lane, _K), axis=1)
    idx_ref[0, 0, :] = idx
    onehot = (lane == idx[:, None]).astype(jnp.float32)
    zq_ref[...] = lax.dot_general(
        onehot, cb, (((1,), (0,)), ((), ())),
        preferred_element_type=jnp.float32,
        precision=lax.Precision.HIGHEST)


def _vq_argmin(flat, cb, fsq, cbsq):
    n = flat.shape[0]
    nblk = n // _BLK
    idx3, zq = pl.pallas_call(
        _vq_body,
        grid=(nblk,),
        in_specs=[
            pl.BlockSpec((_BLK, _D), lambda i: (i, 0)),
            pl.BlockSpec((_K, _D), lambda i: (0, 0)),
            pl.BlockSpec((_BLK, 1), lambda i: (i, 0)),
            pl.BlockSpec((1, _K), lambda i: (0, 0)),
        ],
        out_specs=[
            pl.BlockSpec((1, 1, _BLK), lambda i: (i, 0, 0)),
            pl.BlockSpec((_BLK, _D), lambda i: (i, 0)),
        ],
        out_shape=[
            jax.ShapeDtypeStruct((nblk, 1, _BLK), jnp.int32),
            jax.ShapeDtypeStruct((n, _D), jnp.float32),
        ],
    )(flat, cb, fsq, cbsq)
    return idx3.reshape(-1), zq


def _loss_body(flat_ref, zq_ref, loss_ref):
    i = pl.program_id(0)
    d = zq_ref[...] - flat_ref[...]
    part = jnp.sum(d * d)

    @pl.when(i == 0)
    def _():
        loss_ref[0, 0] = 0.0

    loss_ref[0, 0] += part


def _vq_loss(flat, zq_sc):
    n = flat.shape[0]
    nblk = n // _BLK
    acc = pl.pallas_call(
        _loss_body,
        grid=(nblk,),
        in_specs=[
            pl.BlockSpec((_BLK, _D), lambda i: (i, 0)),
            pl.BlockSpec((_BLK, _D), lambda i: (i, 0)),
        ],
        out_specs=pl.BlockSpec((1, 1), lambda i: (0, 0),
                               memory_space=pltpu.SMEM),
        out_shape=jax.ShapeDtypeStruct((1, 1), jnp.float32),
    )(flat, zq_sc)
    return acc[0, 0]


def _sc_gather(cb, idx):
    n = idx.shape[0]
    bpw = n // _NW
    nch = 7
    ch = bpw // nch
    mesh = plsc.VectorSubcoreMesh(core_axis_name="c", subcore_axis_name="s")

    @functools.partial(
        pl.kernel,
        mesh=mesh,
        out_type=jax.ShapeDtypeStruct((n, _D), jnp.float32),
        scratch_types=[
            pltpu.VMEM((bpw,), jnp.int32),
            pltpu.VMEM((2, ch, _D), jnp.float32),
            pltpu.SemaphoreType.DMA,
            pltpu.SemaphoreType.DMA,
            pltpu.SemaphoreType.DMA,
        ],
    )
    def gather_k(table_hbm, idx_hbm, out_hbm, idx_v, rows_v, gsem, ssem, isem):
        wid = lax.axis_index("s") * _NC + lax.axis_index("c")
        base = wid * bpw
        pltpu.async_copy(idx_hbm.at[pl.ds(base, bpw)], idx_v, isem).wait()
        pltpu.async_copy(table_hbm.at[idx_v.at[pl.ds(0, ch)]], rows_v.at[0], gsem)
        for j in range(nch):
            pltpu.make_async_copy(table_hbm.at[idx_v.at[pl.ds(j * ch, ch)]],
                                  rows_v.at[j % 2], gsem).wait()
            if j > 0:
                pltpu.make_async_copy(rows_v.at[(j - 1) % 2],
                                      out_hbm.at[pl.ds(base + (j - 1) * ch, ch)],
                                      ssem).wait()
            if j + 1 < nch:
                pltpu.async_copy(
                    table_hbm.at[idx_v.at[pl.ds((j + 1) * ch, ch)]],
                    rows_v.at[(j + 1) % 2], gsem)
            pltpu.async_copy(rows_v.at[j % 2],
                             out_hbm.at[pl.ds(base + j * ch, ch)], ssem)
        pltpu.make_async_copy(rows_v.at[(nch - 1) % 2],
                              out_hbm.at[pl.ds(base + (nch - 1) * ch, ch)],
                              ssem).wait()

    return gather_k(cb, idx)


def kernel(x, enc_w1, enc_w2, enc_r1_w1, enc_r1_w2, enc_r2_w1, enc_r2_w2,
           codebook, dec_r1_w1, dec_r1_w2, dec_r2_w1, dec_r2_w2, dec_w1,
           dec_w2):
    ob = lax.optimization_barrier
    xh = jnp.transpose(x, (0, 2, 3, 1))
    h = ob(jax.nn.relu(_conv_nhwc(xh, enc_w1, 2, 1)))
    h = ob(_conv_nhwc(h, enc_w2, 2, 1))
    h = ob(_resblock_nhwc(h, enc_r1_w1, enc_r1_w2))
    z = ob(_resblock_nhwc(h, enc_r2_w1, enc_r2_w2))

    b, hh, ww, c = z.shape
    flat = z.reshape(-1, c)
    fsq = jnp.sum(flat ** 2, axis=1, keepdims=True)
    cbsq = jnp.sum(lax.stop_gradient(codebook) ** 2, axis=1)[None, :]
    idx, zq_tc = _vq_argmin(flat, codebook, fsq, cbsq)
    z_q_sc = _sc_gather(codebook, idx)
    loss = (1.0 + _BETA) * _vq_loss(flat, z_q_sc) / (flat.shape[0] * c)

    zq = jnp.transpose(zq_tc.reshape(b, hh, ww, c), (0, 3, 1, 2))
    d = ob(_resblock(zq, dec_r1_w1, dec_r1_w2))
    d = ob(_resblock(d, dec_r2_w1, dec_r2_w2))
    d = jax.nn.relu(d)
    d = ob(jax.nn.relu(_convT(d, dec_w1, 2, 2)))
    x_tilde = jnp.tanh(_convT(d, dec_w2, 2, 2))
    return x_tilde, loss


# speedup vs baseline: 1.0335x; 1.0302x over previous
import functools

import jax
import jax.numpy as jnp
from jax import lax
from jax.experimental import pallas as pl
from jax.experimental.pallas import tpu as pltpu
from jax.experimental.pallas import tpu_sc as plsc

_BETA = 0.25
_D = 128
_K = 512
_BLK = 1792

_NC = 2
_NS = 16
_NW = _NC * _NS


def _conv(x, w, stride, pad):
    return lax.conv_general_dilated(
        x, w, (stride, stride), ((pad, pad), (pad, pad)),
        dimension_numbers=('NCHW', 'OIHW', 'NCHW'))


def _conv_nhwc(x, w, stride, pad):
    return lax.conv_general_dilated(
        x, jnp.transpose(w, (2, 3, 1, 0)), (stride, stride),
        ((pad, pad), (pad, pad)),
        dimension_numbers=('NHWC', 'HWIO', 'NHWC'))


def _resblock_nhwc(x, w1, w2):
    h = _conv_nhwc(jax.nn.relu(x), w1, 1, 1)
    h = _conv_nhwc(jax.nn.relu(h), w2, 1, 0)
    return x + h


def _convT(x, w, stride, pad_eff):
    return lax.conv_general_dilated(
        x, w, (1, 1), ((pad_eff, pad_eff), (pad_eff, pad_eff)),
        lhs_dilation=(stride, stride),
        dimension_numbers=('NCHW', 'OIHW', 'NCHW'))


def _resblock(x, w1, w2):
    h = _conv(jax.nn.relu(x), w1, 1, 1)
    h = _conv(jax.nn.relu(h), w2, 1, 0)
    return x + h


def _vq_body(flat_ref, cb_ref, fsq_ref, cbsq_ref, idx_ref, zq_ref):
    fb = flat_ref[...]
    cb = cb_ref[...]
    scores = lax.dot_general(
        fb, cb, (((1,), (1,)), ((), ())),
        preferred_element_type=jnp.float32)
    dists = (fsq_ref[...] - 2.0 * scores) + cbsq_ref[...]
    minv = jnp.min(dists, axis=1, keepdims=True)
    lane = lax.broadcasted_iota(jnp.int32, dists.shape, 1)
    idx = jnp.min(jnp.where(dists == minv, lane, _K), axis=1)
    idx_ref[0, 0, :] = idx
    onehot = (lane == idx[:, None]).astype(jnp.float32)
    cb_hi = cb.astype(jnp.bfloat16).astype(jnp.float32)
    cb_lo = cb - cb_hi
    dn = (((1,), (0,)), ((), ()))
    zq_ref[...] = (
        lax.dot_general(onehot, cb_hi, dn, preferred_element_type=jnp.float32)
        + lax.dot_general(onehot, cb_lo, dn, preferred_element_type=jnp.float32))


def _vq_argmin(flat, cb, fsq, cbsq):
    n = flat.shape[0]
    nblk = n // _BLK
    idx3, zq = pl.pallas_call(
        _vq_body,
        grid=(nblk,),
        in_specs=[
            pl.BlockSpec((_BLK, _D), lambda i: (i, 0)),
            pl.BlockSpec((_K, _D), lambda i: (0, 0)),
            pl.BlockSpec((_BLK, 1), lambda i: (i, 0)),
            pl.BlockSpec((1, _K), lambda i: (0, 0)),
        ],
        out_specs=[
            pl.BlockSpec((1, 1, _BLK), lambda i: (i, 0, 0)),
            pl.BlockSpec((_BLK, _D), lambda i: (i, 0)),
        ],
        out_shape=[
            jax.ShapeDtypeStruct((nblk, 1, _BLK), jnp.int32),
            jax.ShapeDtypeStruct((n, _D), jnp.float32),
        ],
    )(flat, cb, fsq, cbsq)
    return idx3.reshape(-1), zq


def _loss_body(flat_ref, zq_ref, loss_ref):
    i = pl.program_id(0)
    d = zq_ref[...] - flat_ref[...]
    part = jnp.sum(d * d)

    @pl.when(i == 0)
    def _():
        loss_ref[0, 0] = 0.0

    loss_ref[0, 0] += part


def _vq_loss(flat, zq_sc):
    n = flat.shape[0]
    nblk = n // _BLK
    acc = pl.pallas_call(
        _loss_body,
        grid=(nblk,),
        in_specs=[
            pl.BlockSpec((_BLK, _D), lambda i: (i, 0)),
            pl.BlockSpec((_BLK, _D), lambda i: (i, 0)),
        ],
        out_specs=pl.BlockSpec((1, 1), lambda i: (0, 0),
                               memory_space=pltpu.SMEM),
        out_shape=jax.ShapeDtypeStruct((1, 1), jnp.float32),
    )(flat, zq_sc)
    return acc[0, 0]


def _sc_gather(cb, idx):
    n = idx.shape[0]
    bpw = n // _NW
    nch = 7
    ch = bpw // nch
    mesh = plsc.VectorSubcoreMesh(core_axis_name="c", subcore_axis_name="s")

    @functools.partial(
        pl.kernel,
        mesh=mesh,
        out_type=jax.ShapeDtypeStruct((n, _D), jnp.float32),
        scratch_types=[
            pltpu.VMEM((bpw,), jnp.int32),
            pltpu.VMEM((2, ch, _D), jnp.float32),
            pltpu.SemaphoreType.DMA,
            pltpu.SemaphoreType.DMA,
            pltpu.SemaphoreType.DMA,
        ],
    )
    def gather_k(table_hbm, idx_hbm, out_hbm, idx_v, rows_v, gsem, ssem, isem):
        wid = lax.axis_index("s") * _NC + lax.axis_index("c")
        base = wid * bpw
        pltpu.async_copy(idx_hbm.at[pl.ds(base, bpw)], idx_v, isem).wait()
        pltpu.async_copy(table_hbm.at[idx_v.at[pl.ds(0, ch)]], rows_v.at[0], gsem)
        for j in range(nch):
            pltpu.make_async_copy(table_hbm.at[idx_v.at[pl.ds(j * ch, ch)]],
                                  rows_v.at[j % 2], gsem).wait()
            if j > 0:
                pltpu.make_async_copy(rows_v.at[(j - 1) % 2],
                                      out_hbm.at[pl.ds(base + (j - 1) * ch, ch)],
                                      ssem).wait()
            if j + 1 < nch:
                pltpu.async_copy(
                    table_hbm.at[idx_v.at[pl.ds((j + 1) * ch, ch)]],
                    rows_v.at[(j + 1) % 2], gsem)
            pltpu.async_copy(rows_v.at[j % 2],
                             out_hbm.at[pl.ds(base + j * ch, ch)], ssem)
        pltpu.make_async_copy(rows_v.at[(nch - 1) % 2],
                              out_hbm.at[pl.ds(base + (nch - 1) * ch, ch)],
                              ssem).wait()

    return gather_k(cb, idx)


def kernel(x, enc_w1, enc_w2, enc_r1_w1, enc_r1_w2, enc_r2_w1, enc_r2_w2,
           codebook, dec_r1_w1, dec_r1_w2, dec_r2_w1, dec_r2_w2, dec_w1,
           dec_w2):
    ob = lax.optimization_barrier
    xh = ob(jnp.transpose(x, (0, 2, 3, 1)))
    h = ob(jax.nn.relu(_conv_nhwc(xh, enc_w1, 2, 1)))
    h = ob(_conv_nhwc(h, enc_w2, 2, 1))
    h = ob(_resblock_nhwc(h, enc_r1_w1, enc_r1_w2))
    z = ob(_resblock_nhwc(h, enc_r2_w1, enc_r2_w2))

    b, hh, ww, c = z.shape
    flat = z.reshape(-1, c)
    fsq = jnp.sum(flat ** 2, axis=1, keepdims=True)
    cbsq = jnp.sum(lax.stop_gradient(codebook) ** 2, axis=1)[None, :]
    idx, zq_tc = _vq_argmin(flat, codebook, fsq, cbsq)
    z_q_sc = _sc_gather(codebook, idx)
    loss = (1.0 + _BETA) * _vq_loss(flat, z_q_sc) / (flat.shape[0] * c)

    zq = jnp.transpose(zq_tc.reshape(b, hh, ww, c), (0, 3, 1, 2))
    d = ob(_resblock(zq, dec_r1_w1, dec_r1_w2))
    d = ob(_resblock(d, dec_r2_w1, dec_r2_w2))
    d = jax.nn.relu(d)
    d = ob(jax.nn.relu(_convT(d, dec_w1, 2, 2)))
    x_tilde = jnp.tanh(_convT(d, dec_w2, 2, 2))
    return x_tilde, loss


# speedup vs baseline: 1.0339x; 1.0004x over previous
import functools

import jax
import jax.numpy as jnp
from jax import lax
from jax.experimental import pallas as pl
from jax.experimental.pallas import tpu as pltpu
from jax.experimental.pallas import tpu_sc as plsc

_BETA = 0.25
_D = 128
_K = 512
_BLK = 1792

_NC = 2
_NS = 16
_NW = _NC * _NS


def _conv(x, w, stride, pad):
    return lax.conv_general_dilated(
        x, w, (stride, stride), ((pad, pad), (pad, pad)),
        dimension_numbers=('NCHW', 'OIHW', 'NCHW'))


def _conv_nhwc(x, w, stride, pad):
    return lax.conv_general_dilated(
        x, jnp.transpose(w, (2, 3, 1, 0)), (stride, stride),
        ((pad, pad), (pad, pad)),
        dimension_numbers=('NHWC', 'HWIO', 'NHWC'))


def _resblock_nhwc(x, w1, w2):
    h = _conv_nhwc(jax.nn.relu(x), w1, 1, 1)
    h = _conv_nhwc(jax.nn.relu(h), w2, 1, 0)
    return x + h


def _convT(x, w, stride, pad_eff):
    return lax.conv_general_dilated(
        x, w, (1, 1), ((pad_eff, pad_eff), (pad_eff, pad_eff)),
        lhs_dilation=(stride, stride),
        dimension_numbers=('NCHW', 'OIHW', 'NCHW'))


def _resblock(x, w1, w2):
    h = _conv(jax.nn.relu(x), w1, 1, 1)
    h = _conv(jax.nn.relu(h), w2, 1, 0)
    return x + h


def _vq_body(flat_ref, cb_ref, fsq_ref, cbsq_ref, idx_ref, zq_ref):
    fb = flat_ref[...]
    cb = cb_ref[...]
    scores = lax.dot_general(
        fb, cb, (((1,), (1,)), ((), ())),
        preferred_element_type=jnp.float32)
    dists = (fsq_ref[...] - 2.0 * scores) + cbsq_ref[...]
    minv = jnp.min(dists, axis=1, keepdims=True)
    lane = lax.broadcasted_iota(jnp.int32, dists.shape, 1)
    idx = jnp.min(jnp.where(dists == minv, lane, _K), axis=1)
    idx_ref[0, 0, :] = idx
    onehot = (lane == idx[:, None]).astype(jnp.float32)
    cb_hi = cb.astype(jnp.bfloat16).astype(jnp.float32)
    cb_lo = cb - cb_hi
    dn = (((1,), (0,)), ((), ()))
    zq_ref[...] = (
        lax.dot_general(onehot, cb_hi, dn, preferred_element_type=jnp.float32)
        + lax.dot_general(onehot, cb_lo, dn, preferred_element_type=jnp.float32))


def _vq_argmin(flat, cb, fsq, cbsq):
    n = flat.shape[0]
    nblk = n // _BLK
    idx3, zq = pl.pallas_call(
        _vq_body,
        grid=(nblk,),
        in_specs=[
            pl.BlockSpec((_BLK, _D), lambda i: (i, 0)),
            pl.BlockSpec((_K, _D), lambda i: (0, 0)),
            pl.BlockSpec((_BLK, 1), lambda i: (i, 0)),
            pl.BlockSpec((1, _K), lambda i: (0, 0)),
        ],
        out_specs=[
            pl.BlockSpec((1, 1, _BLK), lambda i: (i, 0, 0)),
            pl.BlockSpec((_BLK, _D), lambda i: (i, 0)),
        ],
        out_shape=[
            jax.ShapeDtypeStruct((nblk, 1, _BLK), jnp.int32),
            jax.ShapeDtypeStruct((n, _D), jnp.float32),
        ],
    )(flat, cb, fsq, cbsq)
    return idx3.reshape(-1), zq


def _loss_body(flat_ref, zq_ref, loss_ref):
    i = pl.program_id(0)
    d = zq_ref[...] - flat_ref[...]
    part = jnp.sum(d * d)

    @pl.when(i == 0)
    def _():
        loss_ref[0, 0] = 0.0

    loss_ref[0, 0] += part


def _vq_loss(flat, zq_sc):
    n = flat.shape[0]
    nblk = n // _BLK
    acc = pl.pallas_call(
        _loss_body,
        grid=(nblk,),
        in_specs=[
            pl.BlockSpec((_BLK, _D), lambda i: (i, 0)),
            pl.BlockSpec((_BLK, _D), lambda i: (i, 0)),
        ],
        out_specs=pl.BlockSpec((1, 1), lambda i: (0, 0),
                               memory_space=pltpu.SMEM),
        out_shape=jax.ShapeDtypeStruct((1, 1), jnp.float32),
    )(flat, zq_sc)
    return acc[0, 0]


def _sc_gather(cb, idx):
    n = idx.shape[0]
    bpw = n // _NW
    nch = 7
    ch = bpw // nch
    mesh = plsc.VectorSubcoreMesh(core_axis_name="c", subcore_axis_name="s")

    @functools.partial(
        pl.kernel,
        mesh=mesh,
        out_type=jax.ShapeDtypeStruct((n, _D), jnp.float32),
        scratch_types=[
            pltpu.VMEM((bpw,), jnp.int32),
            pltpu.VMEM((2, ch, _D), jnp.float32),
            pltpu.SemaphoreType.DMA,
            pltpu.SemaphoreType.DMA,
            pltpu.SemaphoreType.DMA,
        ],
    )
    def gather_k(table_hbm, idx_hbm, out_hbm, idx_v, rows_v, gsem, ssem, isem):
        wid = lax.axis_index("s") * _NC + lax.axis_index("c")
        base = wid * bpw
        pltpu.async_copy(idx_hbm.at[pl.ds(base, bpw)], idx_v, isem).wait()
        pltpu.async_copy(table_hbm.at[idx_v.at[pl.ds(0, ch)]], rows_v.at[0], gsem)
        for j in range(nch):
            pltpu.make_async_copy(table_hbm.at[idx_v.at[pl.ds(j * ch, ch)]],
                                  rows_v.at[j % 2], gsem).wait()
            if j > 0:
                pltpu.make_async_copy(rows_v.at[(j - 1) % 2],
                                      out_hbm.at[pl.ds(base + (j - 1) * ch, ch)],
                                      ssem).wait()
            if j + 1 < nch:
                pltpu.async_copy(
                    table_hbm.at[idx_v.at[pl.ds((j + 1) * ch, ch)]],
                    rows_v.at[(j + 1) % 2], gsem)
            pltpu.async_copy(rows_v.at[j % 2],
                             out_hbm.at[pl.ds(base + j * ch, ch)], ssem)
        pltpu.make_async_copy(rows_v.at[(nch - 1) % 2],
                              out_hbm.at[pl.ds(base + (nch - 1) * ch, ch)],
                              ssem).wait()

    return gather_k(cb, idx)


def kernel(x, enc_w1, enc_w2, enc_r1_w1, enc_r1_w2, enc_r2_w1, enc_r2_w2,
           codebook, dec_r1_w1, dec_r1_w2, dec_r2_w1, dec_r2_w2, dec_w1,
           dec_w2):
    ob = lax.optimization_barrier
    xc = ob(jnp.transpose(x, (1, 2, 3, 0)))
    h = ob(jax.nn.relu(lax.conv_general_dilated(
        xc, jnp.transpose(enc_w1, (2, 3, 1, 0)), (2, 2), ((1, 1), (1, 1)),
        dimension_numbers=('CHWN', 'HWIO', 'NHWC'))))
    h = ob(_conv_nhwc(h, enc_w2, 2, 1))
    h = ob(_resblock_nhwc(h, enc_r1_w1, enc_r1_w2))
    z = ob(_resblock_nhwc(h, enc_r2_w1, enc_r2_w2))

    b, hh, ww, c = z.shape
    flat = z.reshape(-1, c)
    fsq = jnp.sum(flat ** 2, axis=1, keepdims=True)
    cbsq = jnp.sum(lax.stop_gradient(codebook) ** 2, axis=1)[None, :]
    idx, zq_tc = _vq_argmin(flat, codebook, fsq, cbsq)
    z_q_sc = _sc_gather(codebook, idx)
    loss = (1.0 + _BETA) * _vq_loss(flat, z_q_sc) / (flat.shape[0] * c)

    zq = jnp.transpose(zq_tc.reshape(b, hh, ww, c), (0, 3, 1, 2))
    d = ob(_resblock(zq, dec_r1_w1, dec_r1_w2))
    d = ob(_resblock(d, dec_r2_w1, dec_r2_w2))
    d = jax.nn.relu(d)
    d = ob(jax.nn.relu(_convT(d, dec_w1, 2, 2)))
    x_tilde = jnp.tanh(_convT(d, dec_w2, 2, 2))
    return x_tilde, loss


# speedup vs baseline: 1.0341x; 1.0002x over previous
import functools

import jax
import jax.numpy as jnp
from jax import lax
from jax.experimental import pallas as pl
from jax.experimental.pallas import tpu as pltpu
from jax.experimental.pallas import tpu_sc as plsc

_BETA = 0.25
_D = 128
_K = 512
_BLK = 1792

_NC = 2
_NS = 16
_NW = _NC * _NS


def _conv(x, w, stride, pad):
    return lax.conv_general_dilated(
        x, w, (stride, stride), ((pad, pad), (pad, pad)),
        dimension_numbers=('NCHW', 'OIHW', 'NCHW'))


def _conv_nhwc(x, w, stride, pad):
    return lax.conv_general_dilated(
        x, jnp.transpose(w, (2, 3, 1, 0)), (stride, stride),
        ((pad, pad), (pad, pad)),
        dimension_numbers=('NHWC', 'HWIO', 'NHWC'))


def _resblock_nhwc(x, w1, w2):
    h = _conv_nhwc(jax.nn.relu(x), w1, 1, 1)
    h = _conv_nhwc(jax.nn.relu(h), w2, 1, 0)
    return x + h


def _convT(x, w, stride, pad_eff):
    return lax.conv_general_dilated(
        x, w, (1, 1), ((pad_eff, pad_eff), (pad_eff, pad_eff)),
        lhs_dilation=(stride, stride),
        dimension_numbers=('NCHW', 'OIHW', 'NCHW'))


def _resblock(x, w1, w2):
    h = _conv(jax.nn.relu(x), w1, 1, 1)
    h = _conv(jax.nn.relu(h), w2, 1, 0)
    return x + h


def _vq_body(flat_ref, cb_ref, fsq_ref, cbsq_ref, idx_ref, zq_ref):
    fb = flat_ref[...]
    cb = cb_ref[...]
    scores = lax.dot_general(
        fb, cb, (((1,), (1,)), ((), ())),
        preferred_element_type=jnp.float32)
    dists = (fsq_ref[...] - 2.0 * scores) + cbsq_ref[...]
    minv = jnp.min(dists, axis=1, keepdims=True)
    lane = lax.broadcasted_iota(jnp.int32, dists.shape, 1)
    idx = jnp.min(jnp.where(dists == minv, lane, _K), axis=1)
    idx_ref[0, 0, :] = idx
    onehot = (lane == idx[:, None]).astype(jnp.float32)
    cb_hi = cb.astype(jnp.bfloat16).astype(jnp.float32)
    cb_lo = cb - cb_hi
    dn = (((1,), (0,)), ((), ()))
    zq_ref[...] = (
        lax.dot_general(onehot, cb_hi, dn, preferred_element_type=jnp.float32)
        + lax.dot_general(onehot, cb_lo, dn, preferred_element_type=jnp.float32))


def _vq_argmin(flat, cb, fsq, cbsq):
    n = flat.shape[0]
    nblk = n // _BLK
    idx3, zq = pl.pallas_call(
        _vq_body,
        grid=(nblk,),
        in_specs=[
            pl.BlockSpec((_BLK, _D), lambda i: (i, 0)),
            pl.BlockSpec((_K, _D), lambda i: (0, 0)),
            pl.BlockSpec((_BLK, 1), lambda i: (i, 0)),
            pl.BlockSpec((1, _K), lambda i: (0, 0)),
        ],
        out_specs=[
            pl.BlockSpec((1, 1, _BLK), lambda i: (i, 0, 0)),
            pl.BlockSpec((_BLK, _D), lambda i: (i, 0)),
        ],
        out_shape=[
            jax.ShapeDtypeStruct((nblk, 1, _BLK), jnp.int32),
            jax.ShapeDtypeStruct((n, _D), jnp.float32),
        ],
    )(flat, cb, fsq, cbsq)
    return idx3.reshape(-1), zq


def _loss_body(flat_ref, zq_ref, loss_ref):
    i = pl.program_id(0)
    d = zq_ref[...] - flat_ref[...]
    part = jnp.sum(d * d)

    @pl.when(i == 0)
    def _():
        loss_ref[0, 0] = 0.0

    loss_ref[0, 0] += part


def _vq_loss(flat, zq_sc):
    n = flat.shape[0]
    nblk = n // _BLK
    acc = pl.pallas_call(
        _loss_body,
        grid=(nblk,),
        in_specs=[
            pl.BlockSpec((_BLK, _D), lambda i: (i, 0)),
            pl.BlockSpec((_BLK, _D), lambda i: (i, 0)),
        ],
        out_specs=pl.BlockSpec((1, 1), lambda i: (0, 0),
                               memory_space=pltpu.SMEM),
        out_shape=jax.ShapeDtypeStruct((1, 1), jnp.float32),
    )(flat, zq_sc)
    return acc[0, 0]


def _sc_gather(cb, idx):
    n = idx.shape[0]
    bpw = n // _NW
    nch = 7
    ch = bpw // nch
    mesh = plsc.VectorSubcoreMesh(core_axis_name="c", subcore_axis_name="s")

    @functools.partial(
        pl.kernel,
        mesh=mesh,
        out_type=jax.ShapeDtypeStruct((n, _D), jnp.float32),
        scratch_types=[
            pltpu.VMEM((bpw,), jnp.int32),
            pltpu.VMEM((2, ch, _D), jnp.float32),
            pltpu.SemaphoreType.DMA,
            pltpu.SemaphoreType.DMA,
            pltpu.SemaphoreType.DMA,
        ],
    )
    def gather_k(table_hbm, idx_hbm, out_hbm, idx_v, rows_v, gsem, ssem, isem):
        wid = lax.axis_index("s") * _NC + lax.axis_index("c")
        base = wid * bpw
        pltpu.async_copy(idx_hbm.at[pl.ds(base, bpw)], idx_v, isem).wait()
        pltpu.async_copy(table_hbm.at[idx_v.at[pl.ds(0, ch)]], rows_v.at[0], gsem)
        for j in range(nch):
            pltpu.make_async_copy(table_hbm.at[idx_v.at[pl.ds(j * ch, ch)]],
                                  rows_v.at[j % 2], gsem).wait()
            if j > 0:
                pltpu.make_async_copy(rows_v.at[(j - 1) % 2],
                                      out_hbm.at[pl.ds(base + (j - 1) * ch, ch)],
                                      ssem).wait()
            if j + 1 < nch:
                pltpu.async_copy(
                    table_hbm.at[idx_v.at[pl.ds((j + 1) * ch, ch)]],
                    rows_v.at[(j + 1) % 2], gsem)
            pltpu.async_copy(rows_v.at[j % 2],
                             out_hbm.at[pl.ds(base + j * ch, ch)], ssem)
        pltpu.make_async_copy(rows_v.at[(nch - 1) % 2],
                              out_hbm.at[pl.ds(base + (nch - 1) * ch, ch)],
                              ssem).wait()

    return gather_k(cb, idx)


def kernel(x, enc_w1, enc_w2, enc_r1_w1, enc_r1_w2, enc_r2_w1, enc_r2_w2,
           codebook, dec_r1_w1, dec_r1_w2, dec_r2_w1, dec_r2_w2, dec_w1,
           dec_w2):
    ob = lax.optimization_barrier
    xh = ob(jnp.transpose(x, (0, 2, 3, 1)))
    h = ob(jax.nn.relu(_conv_nhwc(xh, enc_w1, 2, 1)))
    h = ob(_conv_nhwc(h, enc_w2, 2, 1))
    h = ob(_resblock_nhwc(h, enc_r1_w1, enc_r1_w2))
    z = ob(_resblock_nhwc(h, enc_r2_w1, enc_r2_w2))

    b, hh, ww, c = z.shape
    flat = z.reshape(-1, c)
    fsq = jnp.sum(flat ** 2, axis=1, keepdims=True)
    cbsq = jnp.sum(lax.stop_gradient(codebook) ** 2, axis=1)[None, :]
    idx, zq_tc = _vq_argmin(flat, codebook, fsq, cbsq)
    z_q_sc = _sc_gather(codebook, idx)
    loss = (1.0 + _BETA) * _vq_loss(flat, z_q_sc) / (flat.shape[0] * c)

    zq = jnp.transpose(zq_tc.reshape(b, hh, ww, c), (0, 3, 1, 2))
    d = ob(_resblock(zq, dec_r1_w1, dec_r1_w2))
    d = ob(_resblock(d, dec_r2_w1, dec_r2_w2))
    d = jax.nn.relu(d)
    d = ob(jax.nn.relu(_convT(d, dec_w1, 2, 2)))
    x_tilde = jnp.tanh(_convT(d, dec_w2, 2, 2))
    return x_tilde, loss
